# Initial kernel scaffold; baseline (speedup 1.0000x reference)
import jax
import jax.numpy as jnp
from jax.experimental import pallas as pl


def kernel(x, rev_w, rev_b, pm_w, pm_b, pm_lng, pm_lnb, s0_g1w, s0_g1b, s0_g2w, s0_g2b, s0_g3w, s0_g3b, s0_g4w, s0_g4b, s0_lng, s0_lnb, s1_g1w, s1_g1b, s1_g2w, s1_g2b, s1_g3w, s1_g3b, s1_g4w, s1_g4b, s1_lng, s1_lnb, w_gate, e_w1, e_b1, e_w2, e_b2):
    raise NotImplementedError("write your pallas kernel here")


# speedup vs baseline: 1.0141x; 1.0141x over previous
import jax
import jax.numpy as jnp
import numpy as np
from jax.experimental import pallas as pl

B, L, C, P = 16, 512, 64, 96
TPL, TOPK, DC, NE, TK, FF, EL = 16, 3, 512, 8, 2, 2048, 2
T = B * C

_PCHOICES = np.array(sorted({max(2, min(L, L // f)) for f in range(1, L // 2 + 1)}),
                     dtype=np.int32)


def _interp_mat(src, dst):
    if src == 1:
        return np.ones((dst, 1), dtype=np.float32)
    pos = np.linspace(0.0, src - 1.0, dst)
    lo = np.floor(pos).astype(np.int64)
    hi = np.minimum(lo + 1, src - 1)
    w = (pos - lo).astype(np.float32)
    M = np.zeros((dst, src), dtype=np.float32)
    M[np.arange(dst), lo] += 1.0 - w
    M[np.arange(dst), hi] += w
    return M


def _layernorm(x, g, b):
    m = x.mean(-1, keepdims=True)
    v = ((x - m) ** 2).mean(-1, keepdims=True)
    return (x - m) / jnp.sqrt(v + 1e-5) * g + b


def _mk_ppm_branch(per):
    def fn(x, pm_w, pm_b):
        n_p = -(-L // per)
        pad = n_p * per - L
        xp = jnp.pad(x, ((0, 0), (0, 0), (0, pad)))
        xp = xp.reshape(B, C, n_p, per)
        M = jnp.asarray(_interp_mat(per, TPL))
        z = jnp.einsum('bcnp,tp->bcnt', xp, M)
        z = jax.nn.gelu(z @ pm_w + pm_b)
        Mb = jnp.asarray(_interp_mat(TPL, per))
        z = jnp.einsum('bcnt,pt->bcnp', z, Mb)
        return z.reshape(B, C, n_p * per)[:, :, :L]
    return fn


_BRANCHES = [_mk_ppm_branch(int(per)) for per in _PCHOICES]


def _ppm(x, pm_w, pm_b, pm_lng, pm_lnb, periods, freqs):
    amp = jnp.abs(jnp.fft.rfft(x, axis=-1)).mean(axis=(0, 1))
    w = jax.nn.softmax(amp[jnp.asarray(freqs)])
    choices = jnp.asarray(_PCHOICES)
    outs = []
    for k in range(TOPK):
        bi = jnp.searchsorted(choices, periods[k])
        outs.append(jax.lax.switch(bi, _BRANCHES, x, pm_w, pm_b))
    out = sum(w[i] * outs[i] for i in range(TOPK))
    return _layernorm(out, pm_lng, pm_lnb)


def _periods_of(x):
    xt = jnp.swapaxes(x, 1, 2)
    amp = jnp.abs(jnp.fft.rfft(xt, axis=-1)).mean(axis=(0, 1))
    amp = amp.at[0].set(0.0)
    idx = jnp.argsort(amp)[::-1][:TOPK]
    freqs = jnp.maximum(1, idx)
    periods = jnp.maximum(2, jnp.minimum(L, L // freqs))
    return periods, freqs


def _star2_body(x_ref, *refs):
    (g1w0, g1b0, g2w0, g2b0, g3w0, g3b0, g4w0, g4b0, lng0, lnb0,
     g1w1, g1b1, g2w1, g2b1, g3w1, g3b1, g4w1, g4b1, lng1, lnb1,
     out_ref) = refs
    h = x_ref[0]
    layers = ((g1w0, g1b0, g2w0, g2b0, g3w0, g3b0, g4w0, g4b0, lng0, lnb0),
              (g1w1, g1b1, g2w1, g2b1, g3w1, g3b1, g4w1, g4b1, lng1, lnb1))
    for g1w, g1b, g2w, g2b, g3w, g3b, g4w, g4b, lng, lnb in layers:
        g = jax.nn.gelu(jnp.dot(h, g1w[...], preferred_element_type=jnp.float32)
                        + g1b[...])
        g = jnp.dot(g, g2w[...], preferred_element_type=jnp.float32) + g2b[...]
        mx = jnp.max(g, axis=0, keepdims=True)
        ex = jnp.exp(g - mx)
        wgt = ex / jnp.sum(ex, axis=0, keepdims=True)
        core = jnp.sum(g * wgt, axis=0, keepdims=True)
        o = (jnp.dot(h, g3w[:L, :], preferred_element_type=jnp.float32)
             + jnp.dot(core, g3w[L:, :], preferred_element_type=jnp.float32)
             + g3b[...])
        o = jax.nn.gelu(o)
        o = jnp.dot(o, g4w[...], preferred_element_type=jnp.float32) + g4b[...]
        h = _layernorm(h + o, lng[...], lnb[...])
    out_ref[0] = h


def _star2(xt, wts):
    full = lambda shape: pl.BlockSpec(shape, lambda b: (0,) * len(shape))
    in_specs = [pl.BlockSpec((1, C, L), lambda b: (b, 0, 0))]
    for w in wts:
        in_specs.append(full(w.shape))
    return pl.pallas_call(
        _star2_body,
        grid=(B,),
        in_specs=in_specs,
        out_specs=pl.BlockSpec((1, C, L), lambda b: (b, 0, 0)),
        out_shape=jax.ShapeDtypeStruct((B, C, L), jnp.float32),
    )(xt, *wts)


_MOE_TB = 256


def _moe_body(tok_ref, gin_ref, wg_ref, w1_ref, b1_ref, w2_ref, b2_ref,
              out_ref, imp_ref):
    X = tok_ref[...]
    logits = jnp.dot(gin_ref[...], wg_ref[...],
                     preferred_element_type=jnp.float32)
    iota = jax.lax.broadcasted_iota(jnp.int32, logits.shape, 1)
    m1 = jnp.max(logits, axis=1, keepdims=True)
    i1 = jnp.min(jnp.where(logits == m1, iota, NE), axis=1, keepdims=True)
    l2 = jnp.where(iota == i1, -jnp.inf, logits)
    m2 = jnp.max(l2, axis=1, keepdims=True)
    i2 = jnp.min(jnp.where(l2 == m2, iota, NE), axis=1, keepdims=True)
    e2 = jnp.exp(m2 - m1)
    wa = 1.0 / (1.0 + e2)
    wb = e2 / (1.0 + e2)
    gates = (wa * (iota == i1).astype(jnp.float32)
             + wb * (iota == i2).astype(jnp.float32))
    acc = jnp.zeros((_MOE_TB, P), jnp.float32)
    for e in range(NE):
        H = jax.nn.gelu(jnp.dot(X, w1_ref[e],
                                preferred_element_type=jnp.float32)
                        + b1_ref[e][None, :])
        Y = (jnp.dot(H, w2_ref[e], preferred_element_type=jnp.float32)
             + b2_ref[e][None, :])
        acc = acc + gates[:, e:e + 1] * Y
    out_ref[...] = acc

    @pl.when(pl.program_id(0) == 0)
    def _():
        imp_ref[...] = jnp.zeros_like(imp_ref)

    imp_ref[...] += jnp.sum(gates, axis=0, keepdims=True)


def _moe(tok, gin, w_gate, e_w1, e_b1, e_w2, e_b2):
    full = lambda shape: pl.BlockSpec(shape, lambda i: (0,) * len(shape))
    out, imp = pl.pallas_call(
        _moe_body,
        grid=(T // _MOE_TB,),
        in_specs=[
            pl.BlockSpec((_MOE_TB, L), lambda i: (i, 0)),
            pl.BlockSpec((_MOE_TB, L), lambda i: (i, 0)),
            full(w_gate.shape),
            full(e_w1.shape),
            full(e_b1.shape),
            full(e_w2.shape),
            full(e_b2.shape),
        ],
        out_specs=[
            pl.BlockSpec((_MOE_TB, P), lambda i: (i, 0)),
            pl.BlockSpec((1, NE), lambda i: (0, 0)),
        ],
        out_shape=[
            jax.ShapeDtypeStruct((T, P), jnp.float32),
            jax.ShapeDtypeStruct((1, NE), jnp.float32),
        ],
    )(tok, gin, w_gate, e_w1, e_b1, e_w2, e_b2)
    return out, imp[0]


def kernel(x, rev_w, rev_b, pm_w, pm_b, pm_lng, pm_lnb,
           s0_g1w, s0_g1b, s0_g2w, s0_g2b, s0_g3w, s0_g3b, s0_g4w, s0_g4b,
           s0_lng, s0_lnb,
           s1_g1w, s1_g1b, s1_g2w, s1_g2b, s1_g3w, s1_g3b, s1_g4w, s1_g4b,
           s1_lng, s1_lnb,
           w_gate, e_w1, e_b1, e_w2, e_b2):
    periods, freqs = _periods_of(x)
    mean = x.mean(axis=1, keepdims=True)
    std = jnp.sqrt(x.var(axis=1, keepdims=True) + 1e-5)
    xn = (x - mean) / std * rev_w + rev_b
    xt = jnp.swapaxes(xn, 1, 2)

    te = _ppm(xt, pm_w, pm_b, pm_lng, pm_lnb, periods, freqs)

    wts = (s0_g1w, s0_g1b, s0_g2w, s0_g2b, s0_g3w, s0_g3b, s0_g4w, s0_g4b,
           s0_lng, s0_lnb,
           s1_g1w, s1_g1b, s1_g2w, s1_g2b, s1_g3w, s1_g3b, s1_g4w, s1_g4b,
           s1_lng, s1_lnb)
    h = _star2(xt, wts)

    out, imp = _moe(h.reshape(T, L), te.reshape(T, L),
                    w_gate, e_w1, e_b1, e_w2, e_b2)
    moe_loss = imp.var() / (imp.mean() ** 2 + 1e-10)

    y = jnp.swapaxes(out.reshape(B, C, P), 1, 2)
    y = (y - rev_b) / rev_w * std + mean
    return y, moe_loss
